# Initial kernel scaffold; baseline (speedup 1.0000x reference)
import jax
import jax.numpy as jnp
from jax.experimental import pallas as pl


def kernel(x, edge_index, W_pre, b_pre, W_l_first, b_l_first, W_r_first, W_l_h0, b_l_h0, W_r_h0, W_l_h1, b_l_h1, W_r_h1, W_l_h2, b_l_h2, W_r_h2, W_l_out, b_l_out, W_r_out):
    raise NotImplementedError("write your pallas kernel here")


# speedup vs baseline: 2.5005x; 2.5005x over previous
import functools

import jax
import jax.numpy as jnp
from jax import lax
from jax.experimental import pallas as pl
from jax.experimental.pallas import tpu as pltpu
from jax.experimental.pallas import tpu_sc as plsc

N = 10000
E = 320000
D = 128

NC = 2
NS = 16
NW = NC * NS
RANGE = 8 * (-(-N // (8 * NW)))
LAST = N - (NW - 1) * RANGE

CE = 3200
FB = 2048
CAP = E + 2 * FB
GC = 128

_mesh = plsc.VectorSubcoreMesh(core_axis_name="c", subcore_axis_name="s")


def _wid():
    return lax.axis_index("s") * NC + lax.axis_index("c")


@functools.partial(
    pl.kernel,
    mesh=_mesh,
    out_type=[
        jax.ShapeDtypeStruct((NW, CAP), jnp.int32),
        jax.ShapeDtypeStruct((NW, CAP), jnp.int32),
        jax.ShapeDtypeStruct((NW, 16), jnp.int32),
    ],
    scratch_types=[
        pltpu.VMEM((CE,), jnp.int32),
        pltpu.VMEM((CE,), jnp.int32),
        pltpu.VMEM((2 * FB,), jnp.int32),
        pltpu.VMEM((2 * FB,), jnp.int32),
        pltpu.VMEM((16,), jnp.int32),
    ],
    compiler_params=pltpu.CompilerParams(needs_layout_passes=False),
)
def _plan(ei, src_list, dloc_list, counts, src_b, dst_b, csrc, cdlo, cnt_v):
    w = _wid()
    lo = w * RANGE
    hi = jnp.minimum(lo + RANGE, N)

    def chunk_body(ci, carry):
        pltpu.sync_copy(ei.at[0, pl.ds(ci * CE, CE)], src_b)
        pltpu.sync_copy(ei.at[1, pl.ds(ci * CE, CE)], dst_b)

        def vbody(k, c2):
            cnt, nfl = c2
            d = dst_b[pl.ds(k * 16, 16)]
            s = src_b[pl.ds(k * 16, 16)]
            m = (d >= lo) & (d < hi)
            pos = plsc.cumsum(m.astype(jnp.int32))
            addr = cnt + pos - 1
            plsc.store_scatter(csrc, [addr], s, mask=m)
            plsc.store_scatter(cdlo, [addr], d - lo, mask=m)
            cnt = cnt + jnp.max(pos)
            do = cnt >= FB

            @pl.when(do)
            def _():
                pltpu.sync_copy(csrc.at[pl.ds(0, FB)],
                                src_list.at[w, pl.ds(nfl * FB, FB)])
                pltpu.sync_copy(cdlo.at[pl.ds(0, FB)],
                                dloc_list.at[w, pl.ds(nfl * FB, FB)])
                ts = csrc[pl.ds(FB, 16)]
                td = cdlo[pl.ds(FB, 16)]
                csrc[pl.ds(0, 16)] = ts
                cdlo[pl.ds(0, 16)] = td

            cnt = jnp.where(do, cnt - FB, cnt)
            nfl = jnp.where(do, nfl + 1, nfl)
            return (cnt, nfl)

        return lax.fori_loop(0, CE // 16, vbody, carry)

    cnt, nfl = lax.fori_loop(0, E // CE, chunk_body,
                             (jnp.int32(0), jnp.int32(0)))

    lane = lax.broadcasted_iota(jnp.int32, (16,), 0)
    pad_s = w * 97 + lane * 13
    pad_d = jnp.full((16,), RANGE, jnp.int32)
    for t in range(GC // 16):
        csrc[pl.ds(cnt + t * 16, 16)] = pad_s
        cdlo[pl.ds(cnt + t * 16, 16)] = pad_d

    pltpu.sync_copy(csrc.at[pl.ds(0, 2 * FB)],
                    src_list.at[w, pl.ds(nfl * FB, 2 * FB)])
    pltpu.sync_copy(cdlo.at[pl.ds(0, 2 * FB)],
                    dloc_list.at[w, pl.ds(nfl * FB, 2 * FB)])
    total = nfl * FB + cnt
    nch = lax.shift_right_logical(total + (GC - 1), 7)
    cnt_v[...] = jnp.broadcast_to(nch, (16,))
    pltpu.sync_copy(cnt_v, counts.at[w])


@functools.partial(
    pl.kernel,
    mesh=_mesh,
    out_type=jax.ShapeDtypeStruct((N, D), jnp.float32),
    scratch_types=[
        pltpu.VMEM((RANGE + 1, D), jnp.float32),
        pltpu.VMEM((GC,), jnp.int32),
        pltpu.VMEM((GC,), jnp.int32),
        pltpu.VMEM((GC,), jnp.int32),
        pltpu.VMEM((GC,), jnp.int32),
        pltpu.VMEM((GC, D), jnp.float32),
        pltpu.VMEM((GC, D), jnp.float32),
        pltpu.VMEM((16,), jnp.int32),
        pltpu.SemaphoreType.DMA,
        pltpu.SemaphoreType.DMA,
    ],
    compiler_params=pltpu.CompilerParams(needs_layout_passes=False),
)
def _aggregate(h, src_list, dloc_list, counts, out,
               aggr, idx0, idx1, dl0, dl1, rows0, rows1, cnt_v, sem0, sem1):
    w = _wid()
    lo = pl.multiple_of(w * RANGE, 8)

    pltpu.sync_copy(counts.at[w], cnt_v)
    nch = jnp.max(cnt_v[...])

    neg_inf = jnp.full((16,), -jnp.inf, jnp.float32)

    def init_body(i, _):
        aggr[lax.shift_right_logical(i, 3),
             pl.ds(lax.mul(jnp.bitwise_and(i, 7), 16), 16)] = neg_inf
        return 0

    lax.fori_loop(0, (RANGE + 1) * (D // 16), init_body, 0)

    @pl.when(nch > 0)
    def _():
        pltpu.sync_copy(src_list.at[w, pl.ds(0, GC)], idx0)
        pltpu.sync_copy(dloc_list.at[w, pl.ds(0, GC)], dl0)
        pltpu.async_copy(h.at[idx0], rows0, sem0)

    lane = lax.broadcasted_iota(jnp.int32, (16,), 0)
    cols = [jnp.asarray(j * 16, jnp.int32) + lane for j in range(D // 16)]

    bufs = ((idx0, dl0, rows0, sem0), (idx1, dl1, rows1, sem1))

    def compute(g, dl_b, rows_b):
        def qbody(q, _):
            for e in range(16):
                row = q * 16 + e
                dle = plsc.load_gather(
                    dl_b, [jnp.full((16,), row, jnp.int32)])
                for j in range(D // 16):
                    a = plsc.load_gather(aggr, [dle, cols[j]])
                    msg = rows_b[row, pl.ds(j * 16, 16)]
                    plsc.store_scatter(aggr, [dle, cols[j]],
                                       jnp.maximum(a, msg))
            return 0

        lax.fori_loop(0, GC // 16, qbody, 0)

    def pair_body(p, _):
        for b in (0, 1):
            g = p * 2 + b
            idx_c, dl_c, rows_c, sem_c = bufs[b]
            idx_n, dl_n, rows_n, sem_n = bufs[1 - b]

            @pl.when(g < nch)
            def _():
                @pl.when(g + 1 < nch)
                def _():
                    pltpu.sync_copy(src_list.at[w, pl.ds((g + 1) * GC, GC)],
                                    idx_n)
                    pltpu.sync_copy(dloc_list.at[w, pl.ds((g + 1) * GC, GC)],
                                    dl_n)
                    pltpu.async_copy(h.at[idx_n], rows_n, sem_n)

                pltpu.make_async_copy(h.at[idx_c], rows_c, sem_c).wait()
                compute(g, dl_c, rows_c)
        return 0

    lax.fori_loop(0, lax.shift_right_logical(nch + 1, 1), pair_body, 0)

    zero = jnp.zeros((16,), jnp.float32)

    def fin_body(i, _):
        r = lax.shift_right_logical(i, 3)
        cidx = pl.ds(lax.mul(jnp.bitwise_and(i, 7), 16), 16)
        v = aggr[r, cidx]
        aggr[r, cidx] = jnp.where(v == -jnp.inf, zero, v)
        return 0

    lax.fori_loop(0, RANGE * (D // 16), fin_body, 0)

    @pl.when(w < NW - 1)
    def _():
        pltpu.sync_copy(aggr.at[pl.ds(0, RANGE)], out.at[pl.ds(lo, RANGE)])

    @pl.when(w == NW - 1)
    def _():
        pltpu.sync_copy(aggr.at[pl.ds(0, LAST)], out.at[pl.ds(lo, LAST)])


def _pre_body(x_ref, w_ref, b_ref, o_ref):
    o_ref[...] = (
        jnp.dot(x_ref[...], w_ref[...], preferred_element_type=jnp.float32)
        + b_ref[...]
    )


def _conv_body(act, a_ref, h_ref, wl_ref, wr_ref, b_ref, o_ref):
    y = (
        jnp.dot(a_ref[...], wl_ref[...], preferred_element_type=jnp.float32)
        + jnp.dot(h_ref[...], wr_ref[...], preferred_element_type=jnp.float32)
        + b_ref[...]
    )
    if act == "relu":
        y = jnp.maximum(y, 0.0)
    elif act == "l2":
        n = jnp.sqrt(jnp.sum(y * y, axis=-1, keepdims=True))
        y = y / jnp.maximum(n, 1e-12)
    o_ref[...] = y


def _pre(x, wt, b):
    return pl.pallas_call(
        _pre_body,
        out_shape=jax.ShapeDtypeStruct((N, D), jnp.float32),
    )(x, wt, b.reshape(1, D))


def _conv(act, aggr, h, wlt, wrt, b):
    return pl.pallas_call(
        functools.partial(_conv_body, act),
        out_shape=jax.ShapeDtypeStruct((N, D), jnp.float32),
    )(aggr, h, wlt, wrt, b.reshape(1, D))


def kernel(x, edge_index, W_pre, b_pre, W_l_first, b_l_first, W_r_first,
           W_l_h0, b_l_h0, W_r_h0, W_l_h1, b_l_h1, W_r_h1,
           W_l_h2, b_l_h2, W_r_h2, W_l_out, b_l_out, W_r_out):
    src_list, dloc_list, counts = _plan(edge_index)

    wp = W_pre.T
    wlf, wrf = W_l_first.T, W_r_first.T
    wlo, wro = W_l_out.T, W_r_out.T
    hidden = [(W_l_h0.T, b_l_h0, W_r_h0.T),
              (W_l_h1.T, b_l_h1, W_r_h1.T),
              (W_l_h2.T, b_l_h2, W_r_h2.T)]

    outs = []
    h = x
    for i in range(3):
        h = _pre(h, wp, b_pre)
        a = _aggregate(h, src_list, dloc_list, counts)
        h = _conv("relu", a, h, wlf, wrf, b_l_first)
        wlh, blh, wrh = hidden[i]
        a = _aggregate(h, src_list, dloc_list, counts)
        h = _conv("relu", a, h, wlh, wrh, blh)
        a = _aggregate(h, src_list, dloc_list, counts)
        h = _conv("l2", a, h, wlo, wro, b_l_out)
        outs.append(h)
    return tuple(outs)


# speedup vs baseline: 2.5340x; 1.0134x over previous
import functools

import jax
import jax.numpy as jnp
from jax import lax
from jax.experimental import pallas as pl
from jax.experimental.pallas import tpu as pltpu
from jax.experimental.pallas import tpu_sc as plsc

N = 10000
E = 320000
D = 128

NC = 2
NS = 16
NW = NC * NS
RANGE = 8 * (-(-N // (8 * NW)))
LAST = N - (NW - 1) * RANGE

CE = 3200
FB = 2048
CAP = E + 2 * FB
GC = 128

_mesh = plsc.VectorSubcoreMesh(core_axis_name="c", subcore_axis_name="s")


def _wid():
    return lax.axis_index("s") * NC + lax.axis_index("c")


@functools.partial(
    pl.kernel,
    mesh=_mesh,
    out_type=[
        jax.ShapeDtypeStruct((NW, CAP), jnp.int32),
        jax.ShapeDtypeStruct((NW, CAP), jnp.int32),
        jax.ShapeDtypeStruct((NW, 16), jnp.int32),
    ],
    scratch_types=[
        pltpu.VMEM((CE,), jnp.int32),
        pltpu.VMEM((CE,), jnp.int32),
        pltpu.VMEM((2 * FB,), jnp.int32),
        pltpu.VMEM((2 * FB,), jnp.int32),
        pltpu.VMEM((16,), jnp.int32),
    ],
    compiler_params=pltpu.CompilerParams(needs_layout_passes=False),
)
def _plan(ei, src_list, dloc_list, counts, src_b, dst_b, csrc, cdlo, cnt_v):
    w = _wid()
    lo = w * RANGE
    hi = jnp.minimum(lo + RANGE, N)

    def chunk_body(ci, carry):
        pltpu.sync_copy(ei.at[0, pl.ds(ci * CE, CE)], src_b)
        pltpu.sync_copy(ei.at[1, pl.ds(ci * CE, CE)], dst_b)

        def vbody(k, c2):
            cnt, nfl = c2
            d = dst_b[pl.ds(k * 16, 16)]
            s = src_b[pl.ds(k * 16, 16)]
            m = (d >= lo) & (d < hi)
            pos = plsc.cumsum(m.astype(jnp.int32))
            addr = cnt + pos - 1
            plsc.store_scatter(csrc, [addr], s, mask=m)
            plsc.store_scatter(cdlo, [addr], d - lo, mask=m)
            cnt = cnt + jnp.max(pos)
            do = cnt >= FB

            @pl.when(do)
            def _():
                pltpu.sync_copy(csrc.at[pl.ds(0, FB)],
                                src_list.at[w, pl.ds(nfl * FB, FB)])
                pltpu.sync_copy(cdlo.at[pl.ds(0, FB)],
                                dloc_list.at[w, pl.ds(nfl * FB, FB)])
                ts = csrc[pl.ds(FB, 16)]
                td = cdlo[pl.ds(FB, 16)]
                csrc[pl.ds(0, 16)] = ts
                cdlo[pl.ds(0, 16)] = td

            cnt = jnp.where(do, cnt - FB, cnt)
            nfl = jnp.where(do, nfl + 1, nfl)
            return (cnt, nfl)

        return lax.fori_loop(0, CE // 16, vbody, carry)

    cnt, nfl = lax.fori_loop(0, E // CE, chunk_body,
                             (jnp.int32(0), jnp.int32(0)))

    lane = lax.broadcasted_iota(jnp.int32, (16,), 0)
    pad_s = w * 97 + lane * 13
    pad_d = jnp.full((16,), RANGE, jnp.int32)
    for t in range(GC // 16):
        csrc[pl.ds(cnt + t * 16, 16)] = pad_s
        cdlo[pl.ds(cnt + t * 16, 16)] = pad_d

    pltpu.sync_copy(csrc.at[pl.ds(0, 2 * FB)],
                    src_list.at[w, pl.ds(nfl * FB, 2 * FB)])
    pltpu.sync_copy(cdlo.at[pl.ds(0, 2 * FB)],
                    dloc_list.at[w, pl.ds(nfl * FB, 2 * FB)])
    total = nfl * FB + cnt
    nch = lax.shift_right_logical(total + (GC - 1), 7)
    cnt_v[...] = jnp.broadcast_to(nch, (16,))
    pltpu.sync_copy(cnt_v, counts.at[w])


@functools.partial(
    pl.kernel,
    mesh=_mesh,
    out_type=jax.ShapeDtypeStruct((N, D), jnp.float32),
    scratch_types=[
        [pltpu.VMEM(((RANGE + 1) * 16,), jnp.float32)
         for _ in range(D // 16)],
        pltpu.VMEM((GC,), jnp.int32),
        pltpu.VMEM((GC,), jnp.int32),
        pltpu.VMEM((GC,), jnp.int32),
        pltpu.VMEM((GC,), jnp.int32),
        pltpu.VMEM((GC, D), jnp.float32),
        pltpu.VMEM((GC, D), jnp.float32),
        pltpu.VMEM((40, D), jnp.float32),
        pltpu.VMEM((16,), jnp.int32),
        pltpu.SemaphoreType.DMA,
        pltpu.SemaphoreType.DMA,
    ],
    compiler_params=pltpu.CompilerParams(needs_layout_passes=False),
)
def _aggregate(h, src_list, dloc_list, counts, out,
               aggr8, idx0, idx1, dl0, dl1, rows0, rows1, merged, cnt_v,
               sem0, sem1):
    w = _wid()
    lo = pl.multiple_of(w * RANGE, 8)

    pltpu.sync_copy(counts.at[w], cnt_v)
    nch = jnp.max(cnt_v[...])

    neg_inf = jnp.full((16,), -jnp.inf, jnp.float32)

    def init_body(i, _):
        for a in aggr8:
            a[pl.ds(i * 16, 16)] = neg_inf
        return 0

    lax.fori_loop(0, RANGE + 1, init_body, 0)

    @pl.when(nch > 0)
    def _():
        pltpu.sync_copy(src_list.at[w, pl.ds(0, GC)], idx0)
        pltpu.sync_copy(dloc_list.at[w, pl.ds(0, GC)], dl0)
        pltpu.async_copy(h.at[idx0], rows0, sem0)

    lane = lax.broadcasted_iota(jnp.int32, (16,), 0)

    bufs = ((idx0, dl0, rows0, sem0), (idx1, dl1, rows1, sem1))

    def compute(g, dl_b, rows_b):
        def qbody(q, _):
            for e in range(16):
                row = q * 16 + e
                dle = plsc.load_gather(
                    dl_b, [jnp.full((16,), row, jnp.int32)])
                addr = dle * 16 + lane
                for j in range(D // 16):
                    a = plsc.load_gather(aggr8[j], [addr])
                    msg = rows_b[row, pl.ds(j * 16, 16)]
                    plsc.store_scatter(aggr8[j], [addr],
                                       jnp.maximum(a, msg))
            return 0

        lax.fori_loop(0, GC // 16, qbody, 0)

    def pair_body(p, _):
        for b in (0, 1):
            g = p * 2 + b
            idx_c, dl_c, rows_c, sem_c = bufs[b]
            idx_n, dl_n, rows_n, sem_n = bufs[1 - b]

            @pl.when(g < nch)
            def _():
                @pl.when(g + 1 < nch)
                def _():
                    pltpu.sync_copy(src_list.at[w, pl.ds((g + 1) * GC, GC)],
                                    idx_n)
                    pltpu.sync_copy(dloc_list.at[w, pl.ds((g + 1) * GC, GC)],
                                    dl_n)
                    pltpu.async_copy(h.at[idx_n], rows_n, sem_n)

                pltpu.make_async_copy(h.at[idx_c], rows_c, sem_c).wait()
                compute(g, dl_c, rows_c)
        return 0

    lax.fori_loop(0, lax.shift_right_logical(nch + 1, 1), pair_body, 0)

    zero = jnp.zeros((16,), jnp.float32)
    nmb = jnp.where(w == NW - 1, LAST // 40, RANGE // 40)

    def chunk_out(c, _):
        def fin_body(i, _):
            for j, a in enumerate(aggr8):
                v = a[pl.ds((c * 40 + i) * 16, 16)]
                merged[i, pl.ds(j * 16, 16)] = jnp.where(
                    v == -jnp.inf, zero, v)
            return 0

        lax.fori_loop(0, 40, fin_body, 0)
        pltpu.sync_copy(merged, out.at[pl.ds(lo + c * 40, 40)])
        return 0

    lax.fori_loop(0, nmb, chunk_out, 0)


def _pre_body(x_ref, w_ref, b_ref, o_ref):
    o_ref[...] = (
        jnp.dot(x_ref[...], w_ref[...], preferred_element_type=jnp.float32)
        + b_ref[...]
    )


def _conv_body(act, a_ref, h_ref, wl_ref, wr_ref, b_ref, o_ref):
    y = (
        jnp.dot(a_ref[...], wl_ref[...], preferred_element_type=jnp.float32)
        + jnp.dot(h_ref[...], wr_ref[...], preferred_element_type=jnp.float32)
        + b_ref[...]
    )
    if act == "relu":
        y = jnp.maximum(y, 0.0)
    elif act == "l2":
        n = jnp.sqrt(jnp.sum(y * y, axis=-1, keepdims=True))
        y = y / jnp.maximum(n, 1e-12)
    o_ref[...] = y


def _pre(x, wt, b):
    return pl.pallas_call(
        _pre_body,
        out_shape=jax.ShapeDtypeStruct((N, D), jnp.float32),
    )(x, wt, b.reshape(1, D))


def _conv(act, aggr, h, wlt, wrt, b):
    return pl.pallas_call(
        functools.partial(_conv_body, act),
        out_shape=jax.ShapeDtypeStruct((N, D), jnp.float32),
    )(aggr, h, wlt, wrt, b.reshape(1, D))


def kernel(x, edge_index, W_pre, b_pre, W_l_first, b_l_first, W_r_first,
           W_l_h0, b_l_h0, W_r_h0, W_l_h1, b_l_h1, W_r_h1,
           W_l_h2, b_l_h2, W_r_h2, W_l_out, b_l_out, W_r_out):
    src_list, dloc_list, counts = _plan(edge_index)

    wp = W_pre.T
    wlf, wrf = W_l_first.T, W_r_first.T
    wlo, wro = W_l_out.T, W_r_out.T
    hidden = [(W_l_h0.T, b_l_h0, W_r_h0.T),
              (W_l_h1.T, b_l_h1, W_r_h1.T),
              (W_l_h2.T, b_l_h2, W_r_h2.T)]

    outs = []
    h = x
    for i in range(3):
        h = _pre(h, wp, b_pre)
        a = _aggregate(h, src_list, dloc_list, counts)
        h = _conv("relu", a, h, wlf, wrf, b_l_first)
        wlh, blh, wrh = hidden[i]
        a = _aggregate(h, src_list, dloc_list, counts)
        h = _conv("relu", a, h, wlh, wrh, blh)
        a = _aggregate(h, src_list, dloc_list, counts)
        h = _conv("l2", a, h, wlo, wro, b_l_out)
        outs.append(h)
    return tuple(outs)


# speedup vs baseline: 4.5280x; 1.7869x over previous
import functools

import jax
import jax.numpy as jnp
from jax import lax
from jax.experimental import pallas as pl
from jax.experimental.pallas import tpu as pltpu
from jax.experimental.pallas import tpu_sc as plsc

N = 10000
E = 320000
D = 128

NC = 2
NS = 16
NW = NC * NS
RANGE = 8 * (-(-N // (8 * NW)))
LAST = N - (NW - 1) * RANGE

CE = 3200
FB = 2048
CAP = E + 2 * FB
GC = 128

_mesh = plsc.VectorSubcoreMesh(core_axis_name="c", subcore_axis_name="s")


def _wid():
    return lax.axis_index("s") * NC + lax.axis_index("c")


@functools.partial(
    pl.kernel,
    mesh=_mesh,
    out_type=[
        jax.ShapeDtypeStruct((NW, CAP), jnp.int32),
        jax.ShapeDtypeStruct((NW, CAP), jnp.int32),
        jax.ShapeDtypeStruct((NW, 16), jnp.int32),
    ],
    scratch_types=[
        pltpu.VMEM((CE,), jnp.int32),
        pltpu.VMEM((CE,), jnp.int32),
        pltpu.VMEM((2 * FB,), jnp.int32),
        pltpu.VMEM((2 * FB,), jnp.int32),
        pltpu.VMEM((16,), jnp.int32),
    ],
    compiler_params=pltpu.CompilerParams(needs_layout_passes=False),
)
def _plan(ei, src_list, dloc_list, counts, src_b, dst_b, csrc, cdlo, cnt_v):
    w = _wid()
    lo = w * RANGE
    hi = jnp.minimum(lo + RANGE, N)

    def chunk_body(ci, carry):
        pltpu.sync_copy(ei.at[0, pl.ds(ci * CE, CE)], src_b)
        pltpu.sync_copy(ei.at[1, pl.ds(ci * CE, CE)], dst_b)

        def vbody(k, c2):
            cnt, nfl = c2
            d = dst_b[pl.ds(k * 16, 16)]
            s = src_b[pl.ds(k * 16, 16)]
            m = (d >= lo) & (d < hi)
            pos = plsc.cumsum(m.astype(jnp.int32))
            addr = cnt + pos - 1
            plsc.store_scatter(csrc, [addr], s, mask=m)
            plsc.store_scatter(cdlo, [addr], d - lo, mask=m)
            cnt = cnt + jnp.max(pos)
            do = cnt >= FB

            @pl.when(do)
            def _():
                pltpu.sync_copy(csrc.at[pl.ds(0, FB)],
                                src_list.at[w, pl.ds(nfl * FB, FB)])
                pltpu.sync_copy(cdlo.at[pl.ds(0, FB)],
                                dloc_list.at[w, pl.ds(nfl * FB, FB)])
                ts = csrc[pl.ds(FB, 16)]
                td = cdlo[pl.ds(FB, 16)]
                csrc[pl.ds(0, 16)] = ts
                cdlo[pl.ds(0, 16)] = td

            cnt = jnp.where(do, cnt - FB, cnt)
            nfl = jnp.where(do, nfl + 1, nfl)
            return (cnt, nfl)

        return lax.fori_loop(0, CE // 16, vbody, carry)

    cnt, nfl = lax.fori_loop(0, E // CE, chunk_body,
                             (jnp.int32(0), jnp.int32(0)))

    lane = lax.broadcasted_iota(jnp.int32, (16,), 0)
    pad_s = w * 97 + lane * 13
    pad_d = jnp.full((16,), RANGE, jnp.int32)
    for t in range(GC // 16):
        csrc[pl.ds(cnt + t * 16, 16)] = pad_s
        cdlo[pl.ds(cnt + t * 16, 16)] = pad_d

    pltpu.sync_copy(csrc.at[pl.ds(0, 2 * FB)],
                    src_list.at[w, pl.ds(nfl * FB, 2 * FB)])
    pltpu.sync_copy(cdlo.at[pl.ds(0, 2 * FB)],
                    dloc_list.at[w, pl.ds(nfl * FB, 2 * FB)])
    total = nfl * FB + cnt
    nch = lax.shift_right_logical(total + (GC - 1), 7)
    cnt_v[...] = jnp.broadcast_to(nch, (16,))
    pltpu.sync_copy(cnt_v, counts.at[w])


@functools.partial(
    pl.kernel,
    mesh=_mesh,
    out_type=jax.ShapeDtypeStruct((N, D), jnp.float32),
    scratch_types=[
        [pltpu.VMEM(((RANGE + 1) * 16,), jnp.float32)
         for _ in range(D // 16)],
        pltpu.VMEM((GC,), jnp.int32),
        pltpu.VMEM((GC,), jnp.int32),
        pltpu.VMEM((GC,), jnp.int32),
        pltpu.VMEM((GC,), jnp.int32),
        pltpu.VMEM((GC, D), jnp.float32),
        pltpu.VMEM((GC, D), jnp.float32),
        pltpu.VMEM((40, D), jnp.float32),
        pltpu.VMEM((16,), jnp.int32),
        pltpu.SemaphoreType.DMA,
        pltpu.SemaphoreType.DMA,
    ],
    compiler_params=pltpu.CompilerParams(needs_layout_passes=False),
)
def _aggregate(h, src_list, dloc_list, counts, out,
               aggr8, idx0, idx1, dl0, dl1, rows0, rows1, merged, cnt_v,
               sem0, sem1):
    w = _wid()
    lo = pl.multiple_of(w * RANGE, 8)

    pltpu.sync_copy(counts.at[w], cnt_v)
    nch = jnp.max(cnt_v[...])

    neg_inf = jnp.full((16,), -jnp.inf, jnp.float32)

    def init_body(i, _):
        for a in aggr8:
            a[pl.ds(i * 16, 16)] = neg_inf
        return 0

    lax.fori_loop(0, RANGE + 1, init_body, 0)

    @pl.when(nch > 0)
    def _():
        pltpu.sync_copy(src_list.at[w, pl.ds(0, GC)], idx0)
        pltpu.sync_copy(dloc_list.at[w, pl.ds(0, GC)], dl0)
        pltpu.async_copy(h.at[idx0], rows0, sem0)

    lane = lax.broadcasted_iota(jnp.int32, (16,), 0)

    bufs = ((idx0, dl0, rows0, sem0), (idx1, dl1, rows1, sem1))

    def compute(g, dl_b, rows_b):
        def qbody(q, _):
            for e in range(16):
                row = q * 16 + e
                dle = plsc.load_gather(
                    dl_b, [jnp.full((16,), row, jnp.int32)])
                addr = dle * 16 + lane
                avs = [plsc.load_gather(aggr8[j], [addr])
                       for j in range(D // 16)]
                msgs = [rows_b[row, pl.ds(j * 16, 16)]
                        for j in range(D // 16)]
                for j in range(D // 16):
                    plsc.store_scatter(aggr8[j], [addr],
                                       jnp.maximum(avs[j], msgs[j]))
            return 0

        lax.fori_loop(0, GC // 16, qbody, 0)

    def pair_body(p, _):
        for b in (0, 1):
            g = p * 2 + b
            idx_c, dl_c, rows_c, sem_c = bufs[b]
            idx_n, dl_n, rows_n, sem_n = bufs[1 - b]

            @pl.when(g < nch)
            def _():
                @pl.when(g + 1 < nch)
                def _():
                    pltpu.sync_copy(src_list.at[w, pl.ds((g + 1) * GC, GC)],
                                    idx_n)
                    pltpu.sync_copy(dloc_list.at[w, pl.ds((g + 1) * GC, GC)],
                                    dl_n)
                    pltpu.async_copy(h.at[idx_n], rows_n, sem_n)

                pltpu.make_async_copy(h.at[idx_c], rows_c, sem_c).wait()
                compute(g, dl_c, rows_c)
        return 0

    lax.fori_loop(0, lax.shift_right_logical(nch + 1, 1), pair_body, 0)

    zero = jnp.zeros((16,), jnp.float32)
    nmb = jnp.where(w == NW - 1, LAST // 40, RANGE // 40)

    def chunk_out(c, _):
        def fin_body(i, _):
            for j, a in enumerate(aggr8):
                v = a[pl.ds((c * 40 + i) * 16, 16)]
                merged[i, pl.ds(j * 16, 16)] = jnp.where(
                    v == -jnp.inf, zero, v)
            return 0

        lax.fori_loop(0, 40, fin_body, 0)
        pltpu.sync_copy(merged, out.at[pl.ds(lo + c * 40, 40)])
        return 0

    lax.fori_loop(0, nmb, chunk_out, 0)


def _pre_body(x_ref, w_ref, b_ref, o_ref):
    o_ref[...] = (
        jnp.dot(x_ref[...], w_ref[...], preferred_element_type=jnp.float32)
        + b_ref[...]
    )


def _conv_body(act, a_ref, h_ref, wl_ref, wr_ref, b_ref, o_ref):
    y = (
        jnp.dot(a_ref[...], wl_ref[...], preferred_element_type=jnp.float32)
        + jnp.dot(h_ref[...], wr_ref[...], preferred_element_type=jnp.float32)
        + b_ref[...]
    )
    if act == "relu":
        y = jnp.maximum(y, 0.0)
    elif act == "l2":
        n = jnp.sqrt(jnp.sum(y * y, axis=-1, keepdims=True))
        y = y / jnp.maximum(n, 1e-12)
    o_ref[...] = y


def _pre(x, wt, b):
    return pl.pallas_call(
        _pre_body,
        out_shape=jax.ShapeDtypeStruct((N, D), jnp.float32),
    )(x, wt, b.reshape(1, D))


def _conv(act, aggr, h, wlt, wrt, b):
    return pl.pallas_call(
        functools.partial(_conv_body, act),
        out_shape=jax.ShapeDtypeStruct((N, D), jnp.float32),
    )(aggr, h, wlt, wrt, b.reshape(1, D))


def kernel(x, edge_index, W_pre, b_pre, W_l_first, b_l_first, W_r_first,
           W_l_h0, b_l_h0, W_r_h0, W_l_h1, b_l_h1, W_r_h1,
           W_l_h2, b_l_h2, W_r_h2, W_l_out, b_l_out, W_r_out):
    src_list, dloc_list, counts = _plan(edge_index)

    wp = W_pre.T
    wlf, wrf = W_l_first.T, W_r_first.T
    wlo, wro = W_l_out.T, W_r_out.T
    hidden = [(W_l_h0.T, b_l_h0, W_r_h0.T),
              (W_l_h1.T, b_l_h1, W_r_h1.T),
              (W_l_h2.T, b_l_h2, W_r_h2.T)]

    outs = []
    h = x
    for i in range(3):
        h = _pre(h, wp, b_pre)
        a = _aggregate(h, src_list, dloc_list, counts)
        h = _conv("relu", a, h, wlf, wrf, b_l_first)
        wlh, blh, wrh = hidden[i]
        a = _aggregate(h, src_list, dloc_list, counts)
        h = _conv("relu", a, h, wlh, wrh, blh)
        a = _aggregate(h, src_list, dloc_list, counts)
        h = _conv("l2", a, h, wlo, wro, b_l_out)
        outs.append(h)
    return tuple(outs)
